# baseline (speedup 1.0000x reference)
import jax
import jax.numpy as jnp
from jax.experimental import pallas as pl
from jax.experimental.pallas import tpu as pltpu


_TILE = 256


def _embed_kernel(x0_ref, x1_ref, fe_ref, w_ref, inst_ref, supi_ref):
    k = x0_ref.shape[1]
    feat = fe_ref[...]
    inst = jnp.dot(x0_ref[...], feat[:k], preferred_element_type=jnp.float32)
    inst = inst + jnp.dot(x1_ref[...], feat[k:], preferred_element_type=jnp.float32)
    inst_ref[...] = inst
    supi_ref[...] = jnp.dot(inst, w_ref[...], preferred_element_type=jnp.float32)


def _gcn_kernel(a0_ref, a1_ref, a2_ref, fe_ref, w_ref, supi_ref,
                w1_ref, b1_ref, w2_ref, b2_ref, logits_ref, lat_ref):
    f = fe_ref.shape[0]
    supf = jnp.dot(fe_ref[...], w_ref[...], preferred_element_type=jnp.float32)
    supi = supi_ref[...]
    lat = jnp.dot(a0_ref[...], supf, preferred_element_type=jnp.float32)
    lat = lat + jnp.dot(a1_ref[...], supi[:f], preferred_element_type=jnp.float32)
    lat = lat + jnp.dot(a2_ref[...], supi[f:], preferred_element_type=jnp.float32)
    lat = jnp.maximum(lat, 0.0)
    lat_ref[...] = lat
    h = jnp.maximum(
        jnp.dot(lat, w1_ref[...], preferred_element_type=jnp.float32) + b1_ref[...],
        0.0)
    logits_ref[...] = (
        jnp.dot(h, w2_ref[...], preferred_element_type=jnp.float32) + b2_ref[...])


def kernel(X_batch, adj, n_curr_features, feature_embeddings, gcn_weight,
           W1, b1, W2, b2):
    B, F = X_batch.shape
    H = gcn_weight.shape[0]
    C = W2.shape[1]
    Hh = W1.shape[1]
    T = _TILE
    grid = (B // T,)
    parallel = pltpu.CompilerParams(dimension_semantics=("parallel",))

    inst, sup_i = pl.pallas_call(
        _embed_kernel,
        grid=grid,
        in_specs=[
            pl.BlockSpec((T, F // 2), lambda i: (i, 0)),
            pl.BlockSpec((T, F // 2), lambda i: (i, 1)),
            pl.BlockSpec((F, H), lambda i: (0, 0)),
            pl.BlockSpec((H, H), lambda i: (0, 0)),
        ],
        out_specs=[
            pl.BlockSpec((T, H), lambda i: (i, 0)),
            pl.BlockSpec((T, H), lambda i: (i, 0)),
        ],
        out_shape=[
            jax.ShapeDtypeStruct((B, H), jnp.float32),
            jax.ShapeDtypeStruct((B, H), jnp.float32),
        ],
        compiler_params=parallel,
    )(X_batch, X_batch, feature_embeddings, gcn_weight)

    nblk = F // T
    logits, latent = pl.pallas_call(
        _gcn_kernel,
        grid=grid,
        in_specs=[
            pl.BlockSpec((T, F), lambda i: (i + nblk, 0)),
            pl.BlockSpec((T, F), lambda i: (i + nblk, 1)),
            pl.BlockSpec((T, F), lambda i: (i + nblk, 2)),
            pl.BlockSpec((F, H), lambda i: (0, 0)),
            pl.BlockSpec((H, H), lambda i: (0, 0)),
            pl.BlockSpec((B, H), lambda i: (0, 0)),
            pl.BlockSpec((H, Hh), lambda i: (0, 0)),
            pl.BlockSpec((1, Hh), lambda i: (0, 0)),
            pl.BlockSpec((Hh, C), lambda i: (0, 0)),
            pl.BlockSpec((1, C), lambda i: (0, 0)),
        ],
        out_specs=[
            pl.BlockSpec((T, C), lambda i: (i, 0)),
            pl.BlockSpec((T, H), lambda i: (i, 0)),
        ],
        out_shape=[
            jax.ShapeDtypeStruct((B, C), jnp.float32),
            jax.ShapeDtypeStruct((B, H), jnp.float32),
        ],
        compiler_params=parallel,
    )(adj, adj, adj, feature_embeddings, gcn_weight, sup_i,
      W1, b1.reshape(1, Hh), W2, b2.reshape(1, C))

    return (logits, latent, inst)


# speedup vs baseline: 1.1390x; 1.1390x over previous
import jax
import jax.numpy as jnp
from jax.experimental import pallas as pl
from jax.experimental.pallas import tpu as pltpu

_CA = 128
_RA = 12
_CB = 128
_RB = 8
_RLAT = 4
_RLOG = 8
_NTH = 2


def _gcn_fused_kernel(x_hbm, adj_hbm, fe_hbm, w_ref, w1_ref, b1_ref, w2_ref,
                      b2_ref, logits_hbm, lat_hbm, inst_hbm,
                      xbuf, abuf, feat, supf, supi, inst_s, lat_s, log_s,
                      sem_feat, sem_a, sem_b, sem_lat, sem_log, sem_inst):
    B, F = x_hbm.shape
    H = w_ref.shape[0]
    na = B // _CA
    nb = B // _CB

    def a_in(c):
        return pltpu.make_async_copy(
            x_hbm.at[pl.ds(c * _CA, _CA), :], xbuf.at[c % _RA], sem_a.at[c % _RA])

    def b_in(c):
        return pltpu.make_async_copy(
            adj_hbm.at[pl.ds(F + c * _CB, _CB), :], abuf.at[c % _RB],
            sem_b.at[c % _RB])

    def lat_out(c):
        return pltpu.make_async_copy(
            lat_s.at[c % _RLAT], lat_hbm.at[pl.ds(c * _CB, _CB), :],
            sem_lat.at[c % _RLAT])

    def log_out(c):
        return pltpu.make_async_copy(
            log_s.at[c % _RLOG], logits_hbm.at[pl.ds(c * _CB, _CB), :],
            sem_log.at[c % _RLOG])

    cp_feat = pltpu.make_async_copy(fe_hbm.at[pl.ds(0, F), :], feat, sem_feat)
    cp_feat.start()
    for s in range(_RA):
        a_in(s).start(priority=s % _NTH)
    cp_feat.wait()
    supf[...] = jnp.dot(feat[...], w_ref[...], preferred_element_type=jnp.float32)

    for c in range(na):
        a_in(c).wait()
        inst = jnp.dot(xbuf[c % _RA], feat[...],
                       preferred_element_type=jnp.float32)
        inst_s[pl.ds(c * _CA, _CA), :] = inst
        supi[pl.ds(c * _CA, _CA), :] = jnp.dot(
            inst, w_ref[...], preferred_element_type=jnp.float32)
        k = c + _RA
        if k < na:
            a_in(k).start(priority=k % _NTH)
        elif k - na < _RB:
            b_in(k - na).start(priority=k % _NTH)

    cp_inst = pltpu.make_async_copy(inst_s, inst_hbm, sem_inst)
    cp_inst.start()

    for c in range(nb):
        b_in(c).wait()
        a = abuf[c % _RB]
        lat = jnp.dot(a[:, :F], supf[...], preferred_element_type=jnp.float32)
        lat = lat + jnp.dot(a[:, F:], supi[...],
                            preferred_element_type=jnp.float32)
        lat = jnp.maximum(lat, 0.0)
        if c >= _RLAT:
            lat_out(c - _RLAT).wait()
        lat_s[c % _RLAT] = lat
        lat_out(c).start(priority=c % _NTH)
        h = jnp.maximum(
            jnp.dot(lat, w1_ref[...], preferred_element_type=jnp.float32)
            + b1_ref[...], 0.0)
        if c >= _RLOG:
            log_out(c - _RLOG).wait()
        log_s[c % _RLOG] = (
            jnp.dot(h, w2_ref[...], preferred_element_type=jnp.float32)
            + b2_ref[...])
        log_out(c).start(priority=(c + 3) % _NTH)
        k = c + _RB
        if k < nb:
            b_in(k).start(priority=k % _NTH)

    cp_inst.wait()
    for c in range(max(nb - _RLAT, 0), nb):
        lat_out(c).wait()
    for c in range(max(nb - _RLOG, 0), nb):
        log_out(c).wait()


def kernel(X_batch, adj, n_curr_features, feature_embeddings, gcn_weight,
           W1, b1, W2, b2):
    B, F = X_batch.shape
    H = gcn_weight.shape[0]
    C = W2.shape[1]
    Hh = W1.shape[1]

    any_spec = pl.BlockSpec(memory_space=pl.ANY)
    vmem_spec = pl.BlockSpec(memory_space=pltpu.VMEM)

    logits, latent, inst = pl.pallas_call(
        _gcn_fused_kernel,
        in_specs=[any_spec, any_spec, any_spec,
                  vmem_spec, vmem_spec, vmem_spec, vmem_spec, vmem_spec],
        out_specs=[any_spec, any_spec, any_spec],
        out_shape=[
            jax.ShapeDtypeStruct((B, C), jnp.float32),
            jax.ShapeDtypeStruct((B, H), jnp.float32),
            jax.ShapeDtypeStruct((B, H), jnp.float32),
        ],
        scratch_shapes=[
            pltpu.VMEM((_RA, _CA, F), jnp.float32),
            pltpu.VMEM((_RB, _CB, F + B), jnp.float32),
            pltpu.VMEM((F, H), jnp.float32),
            pltpu.VMEM((F, H), jnp.float32),
            pltpu.VMEM((B, H), jnp.float32),
            pltpu.VMEM((B, H), jnp.float32),
            pltpu.VMEM((_RLAT, _CB, H), jnp.float32),
            pltpu.VMEM((_RLOG, _CB, C), jnp.float32),
            pltpu.SemaphoreType.DMA,
            pltpu.SemaphoreType.DMA((_RA,)),
            pltpu.SemaphoreType.DMA((_RB,)),
            pltpu.SemaphoreType.DMA((_RLAT,)),
            pltpu.SemaphoreType.DMA((_RLOG,)),
            pltpu.SemaphoreType.DMA,
        ],
    )(X_batch, adj, feature_embeddings, gcn_weight,
      W1, b1.reshape(1, Hh), W2, b2.reshape(1, C))

    return (logits, latent, inst)
